# Initial kernel scaffold; baseline (speedup 1.0000x reference)
import jax
import jax.numpy as jnp
from jax.experimental import pallas as pl


def kernel(inputs, token_table, pos_table):
    raise NotImplementedError("write your pallas kernel here")


# speedup vs baseline: 1.0344x; 1.0344x over previous
import functools

import jax
import jax.numpy as jnp
from jax import lax
from jax.experimental import pallas as pl
from jax.experimental.pallas import tpu as pltpu
from jax.experimental.pallas import tpu_sc as plsc

_L = 16


def _make_embed_kernel(N, S, D, n_workers, chunk):
    rows_per_w = N // n_workers
    n_chunks = rows_per_w // chunk
    vregs_per_row = D // _L

    mesh = plsc.VectorSubcoreMesh(core_axis_name="c", subcore_axis_name="s")

    @functools.partial(
        pl.kernel,
        mesh=mesh,
        out_type=jax.ShapeDtypeStruct((N, D), jnp.float32),
        scratch_types=[
            pltpu.VMEM((rows_per_w,), jnp.int32),
            pltpu.VMEM((chunk, D), jnp.float32),
            pltpu.VMEM((chunk, D), jnp.float32),
            pltpu.SemaphoreType.DMA,
        ],
    )
    def embed(idx_hbm, tok_hbm, pos_hbm, out_hbm, idx_v, tok_v, pos_v, sem):
        nc = 2
        wid = lax.axis_index("s") * nc + lax.axis_index("c")
        base = wid * rows_per_w
        pos_base = lax.rem(base, S)
        pltpu.sync_copy(idx_hbm.at[pl.ds(base, rows_per_w)], idx_v)

        def chunk_body(ci, _):
            off = ci * chunk
            pltpu.async_copy(
                tok_hbm.at[idx_v.at[pl.ds(off, chunk)]], tok_v, sem
            ).wait()
            pltpu.sync_copy(pos_hbm.at[pl.ds(pos_base + off, chunk)], pos_v)

            def add_row(r, _):
                for c in range(vregs_per_row):
                    sl = pl.ds(c * _L, _L)
                    tok_v[r, sl] = tok_v[r, sl] + pos_v[r, sl]
                return 0

            lax.fori_loop(0, chunk, add_row, 0)
            pltpu.sync_copy(tok_v, out_hbm.at[pl.ds(base + off, chunk)])
            return 0

        lax.fori_loop(0, n_chunks, chunk_body, 0)

    return embed


def kernel(inputs, token_table, pos_table):
    B, S = inputs.shape
    V, D = token_table.shape
    N = B * S
    flat_idx = inputs.reshape(N).astype(jnp.int32)
    embed = _make_embed_kernel(N, S, D, n_workers=32, chunk=64)
    out = embed(flat_idx, token_table, pos_table)
    return out.reshape(B, S, D)
